# baseline (speedup 1.0000x reference)
import functools

import jax
import jax.numpy as jnp
from jax.experimental import pallas as pl
from jax.experimental.pallas import tpu as pltpu


def _conv3x3_kernel(x_ref, w_ref, o_ref, *, H, W):
    L = H * W
    C_in = x_ref.shape[2]
    for b in range(x_ref.shape[0]):
        x = x_ref[b]

        zrow = jnp.zeros((W, C_in), jnp.bfloat16)
        x_up = jnp.concatenate([zrow, x[: L - W]], axis=0)
        x_dn = jnp.concatenate([x[W:], zrow], axis=0)
        p = jnp.concatenate([x_up, x, x_dn], axis=1)

        K3 = 3 * C_in
        zcol = jnp.zeros((1, K3), jnp.bfloat16)
        p_m = jnp.concatenate([zcol, p[: L - 1]], axis=0)
        p_p = jnp.concatenate([p[1:], zcol], axis=0)
        wrow = jax.lax.broadcasted_iota(jnp.int32, (L, K3), 0) % W
        p_m = jnp.where(wrow == 0, jnp.bfloat16(0), p_m)
        p_p = jnp.where(wrow == W - 1, jnp.bfloat16(0), p_p)

        acc = jnp.dot(p, w_ref[1], preferred_element_type=jnp.float32)
        acc = acc + jnp.dot(p_m, w_ref[0], preferred_element_type=jnp.float32)
        acc = acc + jnp.dot(p_p, w_ref[2], preferred_element_type=jnp.float32)
        o_ref[b] = acc


def kernel(x_nchw, w_oihw):
    N, C_in, H, W = x_nchw.shape
    C_out, C_in_w, KH, KW = w_oihw.shape
    assert C_in == C_in_w and KH == KW == 3
    L = H * W

    x_t = jnp.transpose(x_nchw.reshape(N, C_in, L), (0, 2, 1))
    x_t = x_t.astype(jnp.bfloat16)
    w2 = jnp.transpose(w_oihw, (3, 2, 1, 0)).reshape(KW, KH * C_in, C_out)
    w2 = w2.astype(jnp.bfloat16)

    B = 4 if N % 4 == 0 else 1
    body = functools.partial(_conv3x3_kernel, H=H, W=W)
    out_t = pl.pallas_call(
        body,
        out_shape=jax.ShapeDtypeStruct((N, L, C_out), jnp.float32),
        grid_spec=pltpu.PrefetchScalarGridSpec(
            num_scalar_prefetch=0,
            grid=(N // B,),
            in_specs=[
                pl.BlockSpec((B, L, C_in), lambda n: (n, 0, 0)),
                pl.BlockSpec((KW, KH * C_in, C_out), lambda n: (0, 0, 0)),
            ],
            out_specs=pl.BlockSpec((B, L, C_out), lambda n: (n, 0, 0)),
        ),
        compiler_params=pltpu.CompilerParams(
            dimension_semantics=("parallel",)),
    )(x_t, w2)
    return jnp.transpose(out_t.reshape(N, H, W, C_out), (0, 3, 1, 2))


# speedup vs baseline: 1.1598x; 1.1598x over previous
import functools

import jax
import jax.numpy as jnp
from jax.experimental import pallas as pl
from jax.experimental.pallas import tpu as pltpu


def _conv3x3_kernel(x_ref, w_ref, o_ref, *, H, W):
    L = H * W
    C_in = x_ref.shape[2]
    K3 = 3 * C_in
    for b in range(x_ref.shape[0]):
        x = x_ref[b].astype(jnp.bfloat16)

        zrow = jnp.zeros((W, C_in), jnp.bfloat16)
        x_up = jnp.concatenate([zrow, x[: L - W]], axis=0)
        x_dn = jnp.concatenate([x[W:], zrow], axis=0)
        p = jnp.concatenate([x_up, x, x_dn], axis=1)

        zcol = jnp.zeros((1, K3), jnp.bfloat16)
        p_m = jnp.concatenate([zcol, p[: L - 1]], axis=0)
        p_p = jnp.concatenate([p[1:], zcol], axis=0)
        wrow = jax.lax.broadcasted_iota(jnp.int32, (L, K3), 0) % W
        p_m = jnp.where(wrow == 0, jnp.bfloat16(0), p_m)
        p_p = jnp.where(wrow == W - 1, jnp.bfloat16(0), p_p)

        acc = jnp.dot(p, w_ref[1], preferred_element_type=jnp.float32)
        acc = acc + jnp.dot(p_m, w_ref[0], preferred_element_type=jnp.float32)
        acc = acc + jnp.dot(p_p, w_ref[2], preferred_element_type=jnp.float32)
        o_ref[b] = acc


def kernel(x_nchw, w_oihw):
    N, C_in, H, W = x_nchw.shape
    C_out, C_in_w, KH, KW = w_oihw.shape
    assert C_in == C_in_w and KH == KW == 3
    L = H * W

    x_t = jnp.transpose(x_nchw.reshape(N, C_in, L), (0, 2, 1))
    w2 = jnp.transpose(w_oihw, (3, 2, 1, 0)).reshape(KW, KH * C_in, C_out)
    w2 = w2.astype(jnp.bfloat16)

    B = 4 if N % 4 == 0 else 1
    body = functools.partial(_conv3x3_kernel, H=H, W=W)
    out_t = pl.pallas_call(
        body,
        out_shape=jax.ShapeDtypeStruct((N, L, C_out), x_nchw.dtype),
        grid_spec=pltpu.PrefetchScalarGridSpec(
            num_scalar_prefetch=0,
            grid=(N // B,),
            in_specs=[
                pl.BlockSpec((B, L, C_in), lambda n: (n, 0, 0)),
                pl.BlockSpec((KW, KH * C_in, C_out), lambda n: (0, 0, 0)),
            ],
            out_specs=pl.BlockSpec((B, L, C_out), lambda n: (n, 0, 0)),
        ),
        compiler_params=pltpu.CompilerParams(
            dimension_semantics=("parallel",)),
    )(x_t, w2)
    return jnp.transpose(out_t.reshape(N, H, W, C_out), (0, 3, 1, 2))


# speedup vs baseline: 1.1623x; 1.0021x over previous
import functools

import jax
import jax.numpy as jnp
from jax.experimental import pallas as pl
from jax.experimental.pallas import tpu as pltpu


def _conv3x3_kernel(x_ref, w_ref, o_ref, *, H, W):
    L = H * W
    C_in = x_ref.shape[2]
    K3 = 3 * C_in
    for b in range(x_ref.shape[0]):
        x = x_ref[b].astype(jnp.bfloat16)

        zrow = jnp.zeros((W, C_in), jnp.bfloat16)
        x_up = jnp.concatenate([zrow, x[: L - W]], axis=0)
        x_dn = jnp.concatenate([x[W:], zrow], axis=0)
        p = jnp.concatenate([x_up, x, x_dn], axis=1)

        zcol = jnp.zeros((1, K3), jnp.bfloat16)
        p_m = jnp.concatenate([zcol, p[: L - 1]], axis=0)
        p_p = jnp.concatenate([p[1:], zcol], axis=0)
        wrow = jax.lax.broadcasted_iota(jnp.int32, (L, K3), 0) % W
        p_m = jnp.where(wrow == 0, jnp.bfloat16(0), p_m)
        p_p = jnp.where(wrow == W - 1, jnp.bfloat16(0), p_p)

        acc = jnp.dot(p, w_ref[1], preferred_element_type=jnp.float32)
        acc = acc + jnp.dot(p_m, w_ref[0], preferred_element_type=jnp.float32)
        acc = acc + jnp.dot(p_p, w_ref[2], preferred_element_type=jnp.float32)
        o_ref[b] = acc


def kernel(x_nchw, w_oihw):
    N, C_in, H, W = x_nchw.shape
    C_out, C_in_w, KH, KW = w_oihw.shape
    assert C_in == C_in_w and KH == KW == 3
    L = H * W

    x_t = jnp.transpose(x_nchw.reshape(N, C_in, L), (0, 2, 1))
    w2 = jnp.transpose(w_oihw, (3, 2, 1, 0)).reshape(KW, KH * C_in, C_out)
    w2 = w2.astype(jnp.bfloat16)

    B = 2 if N % 2 == 0 else 1
    body = functools.partial(_conv3x3_kernel, H=H, W=W)
    out_t = pl.pallas_call(
        body,
        out_shape=jax.ShapeDtypeStruct((N, L, C_out), x_nchw.dtype),
        grid_spec=pltpu.PrefetchScalarGridSpec(
            num_scalar_prefetch=0,
            grid=(N // B,),
            in_specs=[
                pl.BlockSpec((B, L, C_in), lambda n: (n, 0, 0)),
                pl.BlockSpec((KW, KH * C_in, C_out), lambda n: (0, 0, 0)),
            ],
            out_specs=pl.BlockSpec((B, L, C_out), lambda n: (n, 0, 0)),
        ),
        compiler_params=pltpu.CompilerParams(
            dimension_semantics=("parallel",)),
    )(x_t, w2)
    return jnp.transpose(out_t.reshape(N, H, W, C_out), (0, 3, 1, 2))


# speedup vs baseline: 1.1977x; 1.0305x over previous
import functools

import jax
import jax.numpy as jnp
from jax.experimental import pallas as pl
from jax.experimental.pallas import tpu as pltpu


def _conv3x3_kernel(x_ref, w_ref, o_ref, *, H, W):
    L = H * W
    C_in = x_ref.shape[2]
    K3 = 3 * C_in
    for b in range(x_ref.shape[0]):
        x = x_ref[b].astype(jnp.bfloat16)

        zrow = jnp.zeros((W, C_in), jnp.bfloat16)
        x_up = jnp.concatenate([zrow, x[: L - W]], axis=0)
        x_dn = jnp.concatenate([x[W:], zrow], axis=0)
        p = jnp.concatenate([x_up, x, x_dn], axis=1)

        zcol = jnp.zeros((1, K3), jnp.bfloat16)
        p_m = jnp.concatenate([zcol, p[: L - 1]], axis=0)
        p_p = jnp.concatenate([p[1:], zcol], axis=0)
        wrow = jax.lax.broadcasted_iota(jnp.int32, (L, K3), 0) % W
        p_m = jnp.where(wrow == 0, jnp.bfloat16(0), p_m)
        p_p = jnp.where(wrow == W - 1, jnp.bfloat16(0), p_p)

        p_ext = jnp.concatenate([p_m, p, p_p], axis=1)
        o_ref[b] = jnp.dot(p_ext, w_ref[...],
                           preferred_element_type=jnp.float32)


def kernel(x_nchw, w_oihw):
    N, C_in, H, W = x_nchw.shape
    C_out, C_in_w, KH, KW = w_oihw.shape
    assert C_in == C_in_w and KH == KW == 3
    L = H * W

    x_t = jnp.transpose(x_nchw.reshape(N, C_in, L), (0, 2, 1))
    w2 = jnp.transpose(w_oihw, (3, 2, 1, 0)).reshape(KW * KH * C_in, C_out)
    w2 = w2.astype(jnp.bfloat16)

    B = 4 if N % 4 == 0 else 1
    body = functools.partial(_conv3x3_kernel, H=H, W=W)
    out_t = pl.pallas_call(
        body,
        out_shape=jax.ShapeDtypeStruct((N, L, C_out), x_nchw.dtype),
        grid_spec=pltpu.PrefetchScalarGridSpec(
            num_scalar_prefetch=0,
            grid=(N // B,),
            in_specs=[
                pl.BlockSpec((B, L, C_in), lambda n: (n, 0, 0)),
                pl.BlockSpec((KW * KH * C_in, C_out), lambda n: (0, 0)),
            ],
            out_specs=pl.BlockSpec((B, L, C_out), lambda n: (n, 0, 0)),
        ),
        compiler_params=pltpu.CompilerParams(
            dimension_semantics=("parallel",)),
    )(x_t, w2)
    return jnp.transpose(out_t.reshape(N, H, W, C_out), (0, 3, 1, 2))


# speedup vs baseline: 1.1999x; 1.0018x over previous
import functools

import jax
import jax.numpy as jnp
from jax.experimental import pallas as pl
from jax.experimental.pallas import tpu as pltpu


def _conv3x3_kernel(x_ref, w_ref, o_ref, *, H, W):
    L = H * W
    C_in = x_ref.shape[2]
    K3 = 3 * C_in
    rowpos = jax.lax.broadcasted_iota(jnp.int32, (L, 1), 0) % W
    first_col = rowpos == 0
    last_col = rowpos == W - 1
    for b in range(x_ref.shape[0]):
        x = x_ref[b].astype(jnp.bfloat16)

        zrow = jnp.zeros((W, C_in), jnp.bfloat16)
        x_up = jnp.concatenate([zrow, x[: L - W]], axis=0)
        x_dn = jnp.concatenate([x[W:], zrow], axis=0)
        p = jnp.concatenate([x_up, x, x_dn], axis=1)

        zcol = jnp.zeros((1, K3), jnp.bfloat16)
        p_m = jnp.concatenate([zcol, p[: L - 1]], axis=0)
        p_p = jnp.concatenate([p[1:], zcol], axis=0)
        p_m = jnp.where(first_col, jnp.bfloat16(0), p_m)
        p_p = jnp.where(last_col, jnp.bfloat16(0), p_p)

        p_ext = jnp.concatenate([p_m, p, p_p], axis=1)
        o_ref[b] = jnp.dot(p_ext, w_ref[...],
                           preferred_element_type=jnp.float32)


def kernel(x_nchw, w_oihw):
    N, C_in, H, W = x_nchw.shape
    C_out, C_in_w, KH, KW = w_oihw.shape
    assert C_in == C_in_w and KH == KW == 3
    L = H * W

    x_t = jnp.transpose(x_nchw.reshape(N, C_in, L), (0, 2, 1))
    w2 = jnp.transpose(w_oihw, (3, 2, 1, 0)).reshape(KW * KH * C_in, C_out)
    w2 = w2.astype(jnp.bfloat16)

    B = 4 if N % 4 == 0 else 1
    body = functools.partial(_conv3x3_kernel, H=H, W=W)
    out_t = pl.pallas_call(
        body,
        out_shape=jax.ShapeDtypeStruct((N, L, C_out), x_nchw.dtype),
        grid_spec=pltpu.PrefetchScalarGridSpec(
            num_scalar_prefetch=0,
            grid=(N // B,),
            in_specs=[
                pl.BlockSpec((B, L, C_in), lambda n: (n, 0, 0)),
                pl.BlockSpec((KW * KH * C_in, C_out), lambda n: (0, 0)),
            ],
            out_specs=pl.BlockSpec((B, L, C_out), lambda n: (n, 0, 0)),
        ),
        compiler_params=pltpu.CompilerParams(
            dimension_semantics=("parallel",)),
    )(x_t, w2)
    return jnp.transpose(out_t.reshape(N, H, W, C_out), (0, 3, 1, 2))
